# baseline (speedup 1.0000x reference)
import functools

import jax
import jax.numpy as jnp
from jax import lax
from jax.experimental import pallas as pl
from jax.experimental.pallas import tpu as pltpu
from jax.experimental.pallas import tpu_sc as plsc

_SCALE = 10.0
_DIM = 64
_NBATCH = 16384
_SEQ = 50
_NC = 2
_NS = 16
_NW = _NC * _NS
_BATCH_PER_W = _NBATCH // _NW
_SUPB = 4
_SUP = _SUPB * _SEQ
_SUPERS = _BATCH_PER_W // _SUPB
_NB = 4
_LOOKAHEAD = 2
_SPLITS = ((0, 128), (128, _SUP - 128))


def _sc_body(w_hbm, x_hbm, out3_hbm, idx_v, g0, g1, g2, g3, *sems):
    gs = sems[:_NB]
    os_ = sems[_NB:]
    bufs = (g0, g1, g2, g3)
    wid = lax.axis_index("s") * _NC + lax.axis_index("c")
    b_base = wid * _BATCH_PER_W

    pltpu.sync_copy(
        x_hbm.at[pl.ds(b_base * _SEQ, _BATCH_PER_W * _SEQ)], idx_v
    )

    def gather(s, b, q):
        off, cnt = _SPLITS[q]
        return pltpu.make_async_copy(
            w_hbm.at[idx_v.at[pl.ds(s * _SUP + off, cnt)]],
            bufs[b].at[pl.ds(off, cnt)],
            gs[b],
        )

    def out_copies(s, b):
        return [
            pltpu.make_async_copy(
                bufs[b].at[pl.ds(k * _SEQ, _SEQ)],
                out3_hbm.at[b_base + s * _SUPB + k],
                os_[b],
            )
            for k in range(_SUPB)
        ]

    def fire(s, b):
        for q in range(len(_SPLITS)):
            gather(s, b, q).start()

    for b in range(_LOOKAHEAD):
        fire(b, b)

    def sup_iter(g, i):
        s = g * _NB + i
        b = i
        b2 = (i + _LOOKAHEAD) % _NB
        for q in range(len(_SPLITS)):
            gather(s, b, q).wait()

        @pl.when(s + _LOOKAHEAD < _SUPERS)
        def _():
            @pl.when(s >= _LOOKAHEAD)
            def _():
                for c in out_copies(s - _NB + _LOOKAHEAD, b2):
                    c.wait()

            fire(s + _LOOKAHEAD, b2)

        def scale_row(r, c2):
            for c in range(_DIM // 16):
                sl = pl.ds(c * 16, 16)
                bufs[b][r, sl] = bufs[b][r, sl] * _SCALE
            return c2

        lax.fori_loop(0, _SUP, scale_row, 0, unroll=8)

        for c in out_copies(s, b):
            c.start()

    def outer(g, carry):
        for i in range(_NB):
            sup_iter(g, i)
        return carry

    lax.fori_loop(0, _SUPERS // _NB, outer, 0)

    for b in range(_NB):
        for c in out_copies(_SUPERS - _NB + b, b):
            c.wait()


@functools.partial(jax.jit, static_argnames=())
def kernel(x, weight):
    idx = x.reshape(-1).astype(jnp.int32)
    mesh = plsc.VectorSubcoreMesh(core_axis_name="c", subcore_axis_name="s")
    return pl.kernel(
        _sc_body,
        mesh=mesh,
        out_type=jax.ShapeDtypeStruct((_NBATCH, _SEQ, _DIM), jnp.float32),
        scratch_types=[
            pltpu.VMEM((_BATCH_PER_W * _SEQ,), jnp.int32),
        ]
        + [pltpu.VMEM((_SUP, _DIM), jnp.float32) for _ in range(_NB)]
        + [pltpu.SemaphoreType.DMA for _ in range(2 * _NB)],
        compiler_params=pltpu.CompilerParams(use_tc_tiling_on_sc=False),
    )(weight, idx)


# speedup vs baseline: 1.8029x; 1.8029x over previous
import functools

import jax
import jax.numpy as jnp
from jax import lax
from jax.experimental import pallas as pl
from jax.experimental.pallas import tpu as pltpu
from jax.experimental.pallas import tpu_sc as plsc

_SCALE = 10.0
_DIM = 64
_NBATCH = 16384
_SEQ = 50
_NC = 2
_NS = 16
_NW = _NC * _NS
_BATCH_PER_W = _NBATCH // _NW
_SUPB = 4
_SUP = _SUPB * _SEQ
_SUPERS = _BATCH_PER_W // _SUPB
_NB = 4
_LOOKAHEAD = 2
_SPLITS = ((0, 128), (128, _SUP - 128))


def _sc_body(w_hbm, x_hbm, out3_hbm, idx_v, g0, g1, g2, g3, *sems):
    gs = sems[:_NB]
    os_ = sems[_NB:]
    bufs = (g0, g1, g2, g3)
    wid = lax.axis_index("s") * _NC + lax.axis_index("c")
    b_base = wid * _BATCH_PER_W

    pltpu.sync_copy(
        x_hbm.at[pl.ds(b_base * _SEQ, _BATCH_PER_W * _SEQ)], idx_v
    )

    def gather(s, b, q):
        off, cnt = _SPLITS[q]
        return pltpu.make_async_copy(
            w_hbm.at[idx_v.at[pl.ds(s * _SUP + off, cnt)]],
            bufs[b].at[pl.ds(off, cnt)],
            gs[b],
        )

    def out_copies(s, b):
        return [
            pltpu.make_async_copy(
                bufs[b].at[pl.ds(k * _SEQ, _SEQ)],
                out3_hbm.at[
                    b_base + s * _SUPB + k, pl.ds(0, _SEQ), pl.ds(0, _DIM)
                ],
                os_[b],
            )
            for k in range(_SUPB)
        ]

    def fire(s, b):
        for q in range(len(_SPLITS)):
            gather(s, b, q).start()

    for b in range(_LOOKAHEAD):
        fire(b, b)

    def sup_iter(g, i):
        s = g * _NB + i
        b = i
        b2 = (i + _LOOKAHEAD) % _NB
        for q in range(len(_SPLITS)):
            gather(s, b, q).wait()

        @pl.when(s + _LOOKAHEAD < _SUPERS)
        def _():
            @pl.when(s >= _LOOKAHEAD)
            def _():
                for c in out_copies(s - _NB + _LOOKAHEAD, b2):
                    c.wait()

            fire(s + _LOOKAHEAD, b2)

        def scale_row(r, c2):
            for c in range(_DIM // 16):
                sl = pl.ds(c * 16, 16)
                bufs[b][r, sl] = bufs[b][r, sl] * _SCALE
            return c2

        lax.fori_loop(0, _SUP, scale_row, 0, unroll=8)

        for c in out_copies(s, b):
            c.start()

    def outer(g, carry):
        for i in range(_NB):
            sup_iter(g, i)
        return carry

    lax.fori_loop(0, _SUPERS // _NB, outer, 0)

    for b in range(_NB):
        for c in out_copies(_SUPERS - _NB + b, b):
            c.wait()


@functools.partial(jax.jit, static_argnames=())
def kernel(x, weight):
    idx = x.reshape(-1).astype(jnp.int32)
    mesh = plsc.VectorSubcoreMesh(core_axis_name="c", subcore_axis_name="s")
    padded = pl.kernel(
        _sc_body,
        mesh=mesh,
        out_type=jax.ShapeDtypeStruct((_NBATCH, 56, 128), jnp.float32),
        scratch_types=[
            pltpu.VMEM((_BATCH_PER_W * _SEQ,), jnp.int32),
        ]
        + [pltpu.VMEM((_SUP, _DIM), jnp.float32) for _ in range(_NB)]
        + [pltpu.SemaphoreType.DMA for _ in range(2 * _NB)],
        compiler_params=pltpu.CompilerParams(use_tc_tiling_on_sc=False),
    )(weight, idx)
    return padded[:, :_SEQ, :_DIM]
